# baseline (speedup 1.0000x reference)
import jax
import jax.numpy as jnp
from jax.experimental import pallas as pl
from jax.experimental.pallas import tpu as pltpu

_N = 100000
_F = 128
_K = 2000
_BLK = 2048
_NPAD = 100352
_G = _NPAD // _BLK


def _score_body(emb_ref, w_ref, out_ref):
    w = w_ref[...]
    norm = jnp.sqrt(jnp.sum(w * w)) + 1e-8
    wt = w.reshape(1, _F)
    s = jax.lax.dot_general(
        wt, emb_ref[...], (((1,), (1,)), ((), ())),
        preferred_element_type=jnp.float32,
    )
    out_ref[...] = (s / norm)[None]


def _scores(node_embs, scorer):
    embs_p = jnp.pad(node_embs, ((0, _NPAD - _N), (0, 0)))
    out = pl.pallas_call(
        _score_body,
        grid=(_G,),
        in_specs=[
            pl.BlockSpec((_BLK, _F), lambda i: (i, 0)),
            pl.BlockSpec((_F, 1), lambda i: (0, 0)),
        ],
        out_specs=pl.BlockSpec((1, 1, _BLK), lambda i: (i, 0, 0)),
        out_shape=jax.ShapeDtypeStruct((_G, 1, _BLK), jnp.float32),
    )(embs_p, scorer)
    return out.reshape(_NPAD)[:_N]


def kernel(node_embs, mask, scorer):
    scores = _scores(node_embs, scorer) + mask
    vals, topk_indices = jax.lax.top_k(scores, _K)
    topk_indices = jnp.clip(topk_indices, 0, _N - 1)
    selected_embs = node_embs[topk_indices]
    weights = jnp.tanh(vals)[:, None]
    return (selected_embs * weights).T
